# initial kernel scaffold (unmeasured)
import jax
import jax.numpy as jnp
from jax import lax
from jax.experimental import pallas as pl
from jax.experimental.pallas import tpu as pltpu

N_DEV = 4
M = 4096
K_SHARD = 1024
N = 8192
M_CHUNK = M // N_DEV
NBLK = 1024
NB = N // NBLK


def kernel(x, w_mat, scale_x, scale_w):
    def body(x_ref, w_ref, sx_ref, sw_ref, out_ref,
             comm_ref, send_sems, recv_sems):
        my = lax.axis_index("i")
        left = lax.rem(my + N_DEV - 1, N_DEV)
        right = lax.rem(my + 1, N_DEV)

        barrier_sem = pltpu.get_barrier_semaphore()
        for nbr in (left, right):
            pl.semaphore_signal(barrier_sem, inc=1, device_id=(nbr,),
                                device_id_type=pl.DeviceIdType.MESH)
        pl.semaphore_wait(barrier_sem, 2)

        scale = sx_ref[0] * sw_ref[0]

        def partial(chunk_idx, col0):
            xc = x_ref[pl.ds(chunk_idx * M_CHUNK, M_CHUNK), :].astype(jnp.bfloat16)
            wb = w_ref[:, pl.ds(col0, NBLK)].astype(jnp.bfloat16)
            return jnp.dot(xc, wb, preferred_element_type=jnp.float32)

        for nb in range(NB):
            col0 = nb * NBLK
            comm_ref[0] = partial(left, col0)
            for s in range(N_DEV - 1):
                send_slot = s % 2
                recv_slot = (s + 1) % 2
                rdma = pltpu.make_async_remote_copy(
                    src_ref=comm_ref.at[send_slot],
                    dst_ref=comm_ref.at[recv_slot],
                    send_sem=send_sems.at[send_slot],
                    recv_sem=recv_sems.at[recv_slot],
                    device_id=(right,),
                    device_id_type=pl.DeviceIdType.MESH,
                )
                rdma.start()
                c = lax.rem(my + 2 * N_DEV - 2 - s, N_DEV)
                p = partial(c, col0)
                rdma.wait()
                if s < N_DEV - 2:
                    comm_ref[recv_slot] = comm_ref[recv_slot] + p
                else:
                    out_ref[:, pl.ds(col0, NBLK)] = (
                        (comm_ref[recv_slot] + p) * scale)

    return pl.pallas_call(
        body,
        out_shape=jax.ShapeDtypeStruct((M_CHUNK, N), jnp.float32),
        in_specs=[
            pl.BlockSpec(memory_space=pltpu.VMEM),
            pl.BlockSpec(memory_space=pltpu.VMEM),
            pl.BlockSpec(memory_space=pltpu.SMEM),
            pl.BlockSpec(memory_space=pltpu.SMEM),
        ],
        out_specs=pl.BlockSpec(memory_space=pltpu.VMEM),
        scratch_shapes=[
            pltpu.VMEM((2, M_CHUNK, NBLK), jnp.float32),
            pltpu.SemaphoreType.DMA((2,)),
            pltpu.SemaphoreType.DMA((2,)),
        ],
        compiler_params=pltpu.CompilerParams(collective_id=0),
    )(x, w_mat, scale_x, scale_w)


# baseline (device time: 1183652 ns/iter reference)
import jax
import jax.numpy as jnp
from jax import lax
from jax.experimental import pallas as pl
from jax.experimental.pallas import tpu as pltpu

N_DEV = 4
M = 4096
K_SHARD = 1024
N = 8192
M_CHUNK = M // N_DEV
NBLK = 1024
NB = N // NBLK


def kernel(x, w_mat, scale_x, scale_w):
    def body(x_ref, w_ref, sx_ref, sw_ref, out_ref,
             comm_ref, ostage_ref, send_sems, recv_sems, out_sems):
        my = lax.axis_index("i")
        left = lax.rem(my + N_DEV - 1, N_DEV)
        right = lax.rem(my + 1, N_DEV)

        barrier_sem = pltpu.get_barrier_semaphore()
        for nbr in (left, right):
            pl.semaphore_signal(barrier_sem, inc=1, device_id=(nbr,),
                                device_id_type=pl.DeviceIdType.MESH)
        pl.semaphore_wait(barrier_sem, 2)

        scale = sx_ref[0] * sw_ref[0]

        def partial(chunk_idx, col0):
            xc = x_ref[pl.ds(chunk_idx * M_CHUNK, M_CHUNK), :].astype(jnp.bfloat16)
            wb = w_ref[:, pl.ds(col0, NBLK)].astype(jnp.bfloat16)
            return jnp.dot(xc, wb, preferred_element_type=jnp.float32)

        out_dmas = []
        for nb in range(NB):
            col0 = nb * NBLK
            comm_ref[0] = partial(left, col0)
            for s in range(N_DEV - 1):
                send_slot = s % 2
                recv_slot = (s + 1) % 2
                rdma = pltpu.make_async_remote_copy(
                    src_ref=comm_ref.at[send_slot],
                    dst_ref=comm_ref.at[recv_slot],
                    send_sem=send_sems.at[send_slot],
                    recv_sem=recv_sems.at[recv_slot],
                    device_id=(right,),
                    device_id_type=pl.DeviceIdType.MESH,
                )
                rdma.start()
                c = lax.rem(my + 2 * N_DEV - 2 - s, N_DEV)
                p = partial(c, col0)
                rdma.wait()
                if s < N_DEV - 2:
                    comm_ref[recv_slot] = comm_ref[recv_slot] + p
                else:
                    oslot = nb % 2
                    if nb >= 2:
                        out_dmas[nb - 2].wait()
                    ostage_ref[oslot] = (comm_ref[recv_slot] + p) * scale
                    dma = pltpu.make_async_copy(
                        ostage_ref.at[oslot],
                        out_ref.at[:, pl.ds(col0, NBLK)],
                        out_sems.at[oslot],
                    )
                    dma.start()
                    out_dmas.append(dma)
        out_dmas[NB - 2].wait()
        out_dmas[NB - 1].wait()

    return pl.pallas_call(
        body,
        out_shape=jax.ShapeDtypeStruct((M_CHUNK, N), jnp.float32),
        in_specs=[
            pl.BlockSpec(memory_space=pltpu.VMEM),
            pl.BlockSpec(memory_space=pltpu.VMEM),
            pl.BlockSpec(memory_space=pltpu.SMEM),
            pl.BlockSpec(memory_space=pltpu.SMEM),
        ],
        out_specs=pl.BlockSpec(memory_space=pl.ANY),
        scratch_shapes=[
            pltpu.VMEM((2, M_CHUNK, NBLK), jnp.float32),
            pltpu.VMEM((2, M_CHUNK, NBLK), jnp.float32),
            pltpu.SemaphoreType.DMA((2,)),
            pltpu.SemaphoreType.DMA((2,)),
            pltpu.SemaphoreType.DMA((2,)),
        ],
        compiler_params=pltpu.CompilerParams(
            collective_id=0,
            vmem_limit_bytes=100 * 1024 * 1024,
        ),
    )(x, w_mat, scale_x, scale_w)


# device time: 624554 ns/iter; 1.8952x vs baseline; 1.8952x over previous
import jax
import jax.numpy as jnp
from jax import lax
from jax.experimental import pallas as pl
from jax.experimental.pallas import tpu as pltpu

N_DEV = 4
M = 4096
K_SHARD = 1024
N = 8192
M_CHUNK = M // N_DEV
NBLK = 1024
NB = N // NBLK


def kernel(x, w_mat, scale_x, scale_w):
    def body(x_ref, w_ref, sx_ref, sw_ref, out_ref,
             commA_ref, commB_ref, ostage_ref,
             sendA_sems, recvA_sems, sendB_sems, recvB_sems, out_sems):
        my = lax.axis_index("i")
        left = lax.rem(my + N_DEV - 1, N_DEV)
        right = lax.rem(my + 1, N_DEV)

        barrier_sem = pltpu.get_barrier_semaphore()
        for nbr in (left, right):
            pl.semaphore_signal(barrier_sem, inc=1, device_id=(nbr,),
                                device_id_type=pl.DeviceIdType.MESH)
        pl.semaphore_wait(barrier_sem, 2)

        scale = sx_ref[0] * sw_ref[0]

        def partial(chunk_idx, col0):
            xc = x_ref[pl.ds(chunk_idx * M_CHUNK, M_CHUNK), :].astype(jnp.bfloat16)
            wb = w_ref[:, pl.ds(col0, NBLK)].astype(jnp.bfloat16)
            return jnp.dot(xc, wb, preferred_element_type=jnp.float32)

        out_dmas = {}

        def emit_output(nb, col0, block):
            oslot = nb % 2
            if nb >= 2:
                out_dmas[nb - 2].wait()
            ostage_ref[oslot] = block * scale
            dma = pltpu.make_async_copy(
                ostage_ref.at[oslot],
                out_ref.at[:, pl.ds(col0, NBLK)],
                out_sems.at[oslot],
            )
            dma.start()
            out_dmas[nb] = dma

        for pair in range(NB // 2):
            nbA = 2 * pair
            nbB = 2 * pair + 1
            colA = nbA * NBLK
            colB = nbB * NBLK
            commA_ref[0] = partial(left, colA)
            commB_ref[0] = partial(right, colB)
            for s in range(N_DEV - 1):
                send_slot = s % 2
                recv_slot = (s + 1) % 2
                rdmaA = pltpu.make_async_remote_copy(
                    src_ref=commA_ref.at[send_slot],
                    dst_ref=commA_ref.at[recv_slot],
                    send_sem=sendA_sems.at[send_slot],
                    recv_sem=recvA_sems.at[recv_slot],
                    device_id=(right,),
                    device_id_type=pl.DeviceIdType.MESH,
                )
                rdmaB = pltpu.make_async_remote_copy(
                    src_ref=commB_ref.at[send_slot],
                    dst_ref=commB_ref.at[recv_slot],
                    send_sem=sendB_sems.at[send_slot],
                    recv_sem=recvB_sems.at[recv_slot],
                    device_id=(left,),
                    device_id_type=pl.DeviceIdType.MESH,
                )
                rdmaA.start()
                rdmaB.start()
                cA = lax.rem(my + 2 * N_DEV - 2 - s, N_DEV)
                cB = lax.rem(my + 2 + s, N_DEV)
                pA = partial(cA, colA)
                pB = partial(cB, colB)
                rdmaA.wait()
                rdmaB.wait()
                if s < N_DEV - 2:
                    commA_ref[recv_slot] = commA_ref[recv_slot] + pA
                    commB_ref[recv_slot] = commB_ref[recv_slot] + pB
                else:
                    emit_output(nbA, colA, commA_ref[recv_slot] + pA)
                    emit_output(nbB, colB, commB_ref[recv_slot] + pB)
        out_dmas[NB - 2].wait()
        out_dmas[NB - 1].wait()

    return pl.pallas_call(
        body,
        out_shape=jax.ShapeDtypeStruct((M_CHUNK, N), jnp.float32),
        in_specs=[
            pl.BlockSpec(memory_space=pltpu.VMEM),
            pl.BlockSpec(memory_space=pltpu.VMEM),
            pl.BlockSpec(memory_space=pltpu.SMEM),
            pl.BlockSpec(memory_space=pltpu.SMEM),
        ],
        out_specs=pl.BlockSpec(memory_space=pl.ANY),
        scratch_shapes=[
            pltpu.VMEM((2, M_CHUNK, NBLK), jnp.float32),
            pltpu.VMEM((2, M_CHUNK, NBLK), jnp.float32),
            pltpu.VMEM((2, M_CHUNK, NBLK), jnp.float32),
            pltpu.SemaphoreType.DMA((2,)),
            pltpu.SemaphoreType.DMA((2,)),
            pltpu.SemaphoreType.DMA((2,)),
            pltpu.SemaphoreType.DMA((2,)),
            pltpu.SemaphoreType.DMA((2,)),
        ],
        compiler_params=pltpu.CompilerParams(
            collective_id=0,
            vmem_limit_bytes=100 * 1024 * 1024,
        ),
    )(x, w_mat, scale_x, scale_w)


# device time: 361652 ns/iter; 3.2729x vs baseline; 1.7269x over previous
import jax
import jax.numpy as jnp
from jax import lax
from jax.experimental import pallas as pl
from jax.experimental.pallas import tpu as pltpu

N_DEV = 4
M = 4096
K_SHARD = 1024
N = 8192
M_CHUNK = M // N_DEV
NBLK = 1024
NB = N // NBLK


def kernel(x, w_mat, scale_x, scale_w):
    def body(x_ref, w_ref, sx_ref, sw_ref, out_ref,
             commA_ref, commB_ref, ostage_ref,
             sendA_sems, recvA_sems, sendB_sems, recvB_sems, out_sems):
        my = lax.axis_index("i")
        left = lax.rem(my + N_DEV - 1, N_DEV)
        right = lax.rem(my + 1, N_DEV)

        barrier_sem = pltpu.get_barrier_semaphore()
        for nbr in (left, right):
            pl.semaphore_signal(barrier_sem, inc=1, device_id=(nbr,),
                                device_id_type=pl.DeviceIdType.MESH)
        pl.semaphore_wait(barrier_sem, 2)

        scale = sx_ref[0] * sw_ref[0]

        def partial(chunk_idx, col0):
            xc = x_ref[pl.ds(chunk_idx * M_CHUNK, M_CHUNK), :].astype(jnp.bfloat16)
            wb = w_ref[:, pl.ds(col0, NBLK)].astype(jnp.bfloat16)
            return jnp.dot(xc, wb, preferred_element_type=jnp.float32)

        out_dmas = {}

        def emit_output(nb, col0, block):
            oslot = nb % 2
            if nb >= 2:
                out_dmas[nb - 2].wait()
            ostage_ref[oslot] = block * scale
            dma = pltpu.make_async_copy(
                ostage_ref.at[oslot],
                out_ref.at[:, pl.ds(col0, NBLK)],
                out_sems.at[oslot],
            )
            dma.start()
            out_dmas[nb] = dma

        for pair in range(NB // 2):
            nbA = 2 * pair
            nbB = 2 * pair + 1
            colA = nbA * NBLK
            colB = nbB * NBLK
            commA_ref[0] = partial(left, colA).astype(jnp.bfloat16)
            commB_ref[0] = partial(right, colB).astype(jnp.bfloat16)
            for s in range(N_DEV - 1):
                send_slot = s % 2
                recv_slot = (s + 1) % 2
                rdmaA = pltpu.make_async_remote_copy(
                    src_ref=commA_ref.at[send_slot],
                    dst_ref=commA_ref.at[recv_slot],
                    send_sem=sendA_sems.at[send_slot],
                    recv_sem=recvA_sems.at[recv_slot],
                    device_id=(right,),
                    device_id_type=pl.DeviceIdType.MESH,
                )
                rdmaB = pltpu.make_async_remote_copy(
                    src_ref=commB_ref.at[send_slot],
                    dst_ref=commB_ref.at[recv_slot],
                    send_sem=sendB_sems.at[send_slot],
                    recv_sem=recvB_sems.at[recv_slot],
                    device_id=(left,),
                    device_id_type=pl.DeviceIdType.MESH,
                )
                rdmaA.start()
                rdmaB.start()
                cA = lax.rem(my + 2 * N_DEV - 2 - s, N_DEV)
                cB = lax.rem(my + 2 + s, N_DEV)
                pA = partial(cA, colA)
                pB = partial(cB, colB)
                rdmaA.wait()
                rdmaB.wait()
                if s < N_DEV - 2:
                    commA_ref[recv_slot] = (
                        commA_ref[recv_slot].astype(jnp.float32) + pA
                    ).astype(jnp.bfloat16)
                    commB_ref[recv_slot] = (
                        commB_ref[recv_slot].astype(jnp.float32) + pB
                    ).astype(jnp.bfloat16)
                else:
                    emit_output(
                        nbA, colA, commA_ref[recv_slot].astype(jnp.float32) + pA)
                    emit_output(
                        nbB, colB, commB_ref[recv_slot].astype(jnp.float32) + pB)
        out_dmas[NB - 2].wait()
        out_dmas[NB - 1].wait()

    return pl.pallas_call(
        body,
        out_shape=jax.ShapeDtypeStruct((M_CHUNK, N), jnp.float32),
        in_specs=[
            pl.BlockSpec(memory_space=pltpu.VMEM),
            pl.BlockSpec(memory_space=pltpu.VMEM),
            pl.BlockSpec(memory_space=pltpu.SMEM),
            pl.BlockSpec(memory_space=pltpu.SMEM),
        ],
        out_specs=pl.BlockSpec(memory_space=pl.ANY),
        scratch_shapes=[
            pltpu.VMEM((2, M_CHUNK, NBLK), jnp.bfloat16),
            pltpu.VMEM((2, M_CHUNK, NBLK), jnp.bfloat16),
            pltpu.VMEM((2, M_CHUNK, NBLK), jnp.float32),
            pltpu.SemaphoreType.DMA((2,)),
            pltpu.SemaphoreType.DMA((2,)),
            pltpu.SemaphoreType.DMA((2,)),
            pltpu.SemaphoreType.DMA((2,)),
            pltpu.SemaphoreType.DMA((2,)),
        ],
        compiler_params=pltpu.CompilerParams(
            collective_id=0,
            vmem_limit_bytes=100 * 1024 * 1024,
        ),
    )(x, w_mat, scale_x, scale_w)


# device time: 341656 ns/iter; 3.4645x vs baseline; 1.0585x over previous
import jax
import jax.numpy as jnp
from jax import lax
from jax.experimental import pallas as pl
from jax.experimental.pallas import tpu as pltpu

N_DEV = 4
M = 4096
K_SHARD = 1024
N = 8192
M_CHUNK = M // N_DEV
NBLK = 1024
NB = N // NBLK
NPAIR = NB // 2
SLOTS = 5


def kernel(x, w_mat, scale_x, scale_w):
    def body(x_ref, w_ref, sx_ref, sw_ref, out_ref,
             commA_ref, commB_ref, ostage_ref,
             sendA_sems, recvA_sems, sendB_sems, recvB_sems, out_sems):
        my = lax.axis_index("i")
        left = lax.rem(my + N_DEV - 1, N_DEV)
        right = lax.rem(my + 1, N_DEV)

        barrier_sem = pltpu.get_barrier_semaphore()
        for nbr in (left, right):
            pl.semaphore_signal(barrier_sem, inc=1, device_id=(nbr,),
                                device_id_type=pl.DeviceIdType.MESH)
        pl.semaphore_wait(barrier_sem, 2)

        scale = sx_ref[0] * sw_ref[0]

        def partial(chunk_idx, col0):
            xc = x_ref[pl.ds(chunk_idx * M_CHUNK, M_CHUNK), :].astype(jnp.bfloat16)
            wb = w_ref[:, pl.ds(col0, NBLK)].astype(jnp.bfloat16)
            return jnp.dot(xc, wb, preferred_element_type=jnp.float32)

        out_dmas = {}

        def emit_output(nb, block):
            oslot = nb % 2
            if nb >= 2:
                out_dmas[nb - 2].wait()
            ostage_ref[oslot] = block * scale
            dma = pltpu.make_async_copy(
                ostage_ref.at[oslot],
                out_ref.at[:, pl.ds(nb * NBLK, NBLK)],
                out_sems.at[oslot],
            )
            dma.start()
            out_dmas[nb] = dma

        commA_ref[0] = partial(left, 0).astype(jnp.bfloat16)
        commB_ref[0] = partial(right, NBLK).astype(jnp.bfloat16)

        for pair in range(NPAIR):
            b = (4 * pair) % SLOTS
            nbA = 2 * pair
            nbB = 2 * pair + 1
            colA = nbA * NBLK
            colB = nbB * NBLK
            prev_slot = (b + 4) % SLOTS
            for s in range(N_DEV - 1):
                send_slot = (b + s) % SLOTS
                recv_slot = (b + s + 1) % SLOTS
                rdmaA = pltpu.make_async_remote_copy(
                    src_ref=commA_ref.at[send_slot],
                    dst_ref=commA_ref.at[recv_slot],
                    send_sem=sendA_sems.at[send_slot],
                    recv_sem=recvA_sems.at[recv_slot],
                    device_id=(right,),
                    device_id_type=pl.DeviceIdType.MESH,
                )
                rdmaB = pltpu.make_async_remote_copy(
                    src_ref=commB_ref.at[send_slot],
                    dst_ref=commB_ref.at[recv_slot],
                    send_sem=sendB_sems.at[send_slot],
                    recv_sem=recvB_sems.at[recv_slot],
                    device_id=(left,),
                    device_id_type=pl.DeviceIdType.MESH,
                )
                rdmaA.start()
                rdmaB.start()
                if s < N_DEV - 2:
                    cA = lax.rem(my + 2 * N_DEV - 2 - s, N_DEV)
                    cB = lax.rem(my + 2 + s, N_DEV)
                    pA = partial(cA, colA)
                    pB = partial(cB, colB)
                    if pair > 0:
                        nb_prev = 2 * (pair - 1) + s
                        ring_prev = commA_ref if s == 0 else commB_ref
                        o = partial(my, nb_prev * NBLK)
                        emit_output(
                            nb_prev,
                            ring_prev[prev_slot].astype(jnp.float32) + o)
                    rdmaA.wait()
                    rdmaB.wait()
                    commA_ref[recv_slot] = (
                        commA_ref[recv_slot].astype(jnp.float32) + pA
                    ).astype(jnp.bfloat16)
                    commB_ref[recv_slot] = (
                        commB_ref[recv_slot].astype(jnp.float32) + pB
                    ).astype(jnp.bfloat16)
                else:
                    if pair < NPAIR - 1:
                        seed_slot = (b + 4) % SLOTS
                        commA_ref[seed_slot] = partial(
                            left, (nbA + 2) * NBLK).astype(jnp.bfloat16)
                        commB_ref[seed_slot] = partial(
                            right, (nbB + 2) * NBLK).astype(jnp.bfloat16)
                        rdmaA.wait()
                        rdmaB.wait()
                    else:
                        oA = partial(my, colA)
                        oB = partial(my, colB)
                        rdmaA.wait()
                        rdmaB.wait()
                        emit_output(
                            nbA, commA_ref[recv_slot].astype(jnp.float32) + oA)
                        emit_output(
                            nbB, commB_ref[recv_slot].astype(jnp.float32) + oB)
        out_dmas[NB - 2].wait()
        out_dmas[NB - 1].wait()

    return pl.pallas_call(
        body,
        out_shape=jax.ShapeDtypeStruct((M_CHUNK, N), jnp.float32),
        in_specs=[
            pl.BlockSpec(memory_space=pltpu.VMEM),
            pl.BlockSpec(memory_space=pltpu.VMEM),
            pl.BlockSpec(memory_space=pltpu.SMEM),
            pl.BlockSpec(memory_space=pltpu.SMEM),
        ],
        out_specs=pl.BlockSpec(memory_space=pl.ANY),
        scratch_shapes=[
            pltpu.VMEM((SLOTS, M_CHUNK, NBLK), jnp.bfloat16),
            pltpu.VMEM((SLOTS, M_CHUNK, NBLK), jnp.bfloat16),
            pltpu.VMEM((2, M_CHUNK, NBLK), jnp.float32),
            pltpu.SemaphoreType.DMA((SLOTS,)),
            pltpu.SemaphoreType.DMA((SLOTS,)),
            pltpu.SemaphoreType.DMA((SLOTS,)),
            pltpu.SemaphoreType.DMA((SLOTS,)),
            pltpu.SemaphoreType.DMA((2,)),
        ],
        compiler_params=pltpu.CompilerParams(
            collective_id=0,
            vmem_limit_bytes=100 * 1024 * 1024,
        ),
    )(x, w_mat, scale_x, scale_w)
